# Initial kernel scaffold; baseline (speedup 1.0000x reference)
import jax
import jax.numpy as jnp
from jax.experimental import pallas as pl


def kernel(x, edge_index, batch, W0, theta0, W1, theta1, Wc, bc):
    raise NotImplementedError("write your pallas kernel here")


# speedup vs baseline: 12.8526x; 12.8526x over previous
import functools

import jax
import jax.numpy as jnp
from jax import lax
from jax.experimental import pallas as pl
from jax.experimental.pallas import tpu as pltpu
from jax.experimental.pallas import tpu_sc as plsc

_N_NODES = 10000
_N_EDGES = 320000
_D_FEAT = 128
_NQ = 16
_N_GRAPHS = 64
_OUT = 10

_NW = 32
_EDGES_PER_W = 10240
_E_PAD = _NW * _EDGES_PER_W
_GROUPS = 10
_GSZ = 1024
_CH = 128
_CH_PER_G = _GSZ // _CH
_NROWS = 10240
_TRASH = _N_NODES
_RPT = _NROWS // 16

_ROW_BLK = 1000


def _sc_agg_body(y_hbm, src_hbm, dst_hbm, z2_hbm, z1_hbm, ones_hbm,
                 agg_out, deg_out,
                 src_v, dst_v, rows_v, ones_v, acc_s, deg_s, sem):
    cid = lax.axis_index("c")
    sid = lax.axis_index("s")
    wid = sid * 2 + cid

    pltpu.sync_copy(z2_hbm, acc_s.at[pl.ds(sid * _RPT, _RPT)])
    pltpu.sync_copy(z1_hbm, deg_s.at[pl.ds(sid * _RPT, _RPT)])
    pltpu.sync_copy(src_hbm.at[wid], src_v)
    pltpu.sync_copy(dst_hbm.at[wid], dst_v)
    pltpu.sync_copy(ones_hbm, ones_v)
    plsc.subcore_barrier()

    def group(g, carry):
        pltpu.async_copy(y_hbm.at[src_v.at[g]], rows_v, sem).wait()

        def chunk(j, c2):
            idx = dst_v.at[g * _CH_PER_G + j]
            pltpu.sync_copy(rows_v.at[pl.ds(j * _CH, _CH)], acc_s.at[idx],
                            add=True)
            pltpu.sync_copy(ones_v, deg_s.at[idx], add=True)
            return c2

        return lax.fori_loop(0, _CH_PER_G, chunk, carry)

    lax.fori_loop(0, _GROUPS, group, 0)
    plsc.subcore_barrier()

    pltpu.sync_copy(acc_s.at[pl.ds(sid * _RPT, _RPT)],
                    agg_out.at[cid, pl.ds(sid * _RPT, _RPT)])
    pltpu.sync_copy(deg_s.at[pl.ds(sid * _RPT, _RPT)],
                    deg_out.at[cid, pl.ds(sid * _RPT, _RPT)])


_sc_agg = pl.kernel(
    _sc_agg_body,
    out_type=(jax.ShapeDtypeStruct((2, _NROWS, _NQ), jnp.float32),
              jax.ShapeDtypeStruct((2, _NROWS), jnp.float32)),
    mesh=plsc.VectorSubcoreMesh(core_axis_name="c", subcore_axis_name="s"),
    scratch_types=[
        pltpu.VMEM((_GROUPS, _GSZ), jnp.int32),
        pltpu.VMEM((_GROUPS * _CH_PER_G, _CH), jnp.int32),
        pltpu.VMEM((_GSZ, _NQ), jnp.float32),
        pltpu.VMEM((_CH,), jnp.float32),
        pltpu.VMEM_SHARED((_NROWS, _NQ), jnp.float32),
        pltpu.VMEM_SHARED((_NROWS,), jnp.float32),
        pltpu.SemaphoreType.DMA,
    ],
    compiler_params=pltpu.CompilerParams(use_tc_tiling_on_sc=False),
)


def _mm0_body(x_ref, w_ref, o_ref):
    o_ref[...] = jnp.dot(x_ref[...], w_ref[...],
                         preferred_element_type=jnp.float32)


def _shift_mat():
    row = lax.broadcasted_iota(jnp.int32, (_NQ, _NQ), 0)
    col = lax.broadcasted_iota(jnp.int32, (_NQ, _NQ), 1)
    return (col == (row + 1) % _NQ).astype(jnp.float32)


def _post(aggp, degp, c, s):
    a = aggp[0] + aggp[1]
    d = degp[0] + degp[1]
    h = jnp.tanh(a / jnp.maximum(d, 1.0))
    S = _shift_mat()
    for dd in range(2):
        h = (c[dd][None, :] * h
             + s[dd][None, :] * jnp.dot(h, S, preferred_element_type=jnp.float32))
    return jnp.where(h >= 0, h, 0.2 * h)


def _mid_body(aggp_ref, degp_ref, c_ref, s_ref, w_ref, o_ref):
    h = _post(aggp_ref[...], degp_ref[...], c_ref[...], s_ref[...])
    o_ref[...] = jnp.dot(h, w_ref[...], preferred_element_type=jnp.float32)


def _fin_body(aggp_ref, degp_ref, c_ref, s_ref, b_ref, wc_ref, bc_ref, o_ref,
              sums, cnts):
    i = pl.program_id(0)
    h = _post(aggp_ref[...], degp_ref[...], c_ref[...], s_ref[...])
    onehot = (lax.broadcasted_iota(jnp.int32, (_N_GRAPHS, _ROW_BLK), 0)
              == jnp.broadcast_to(b_ref[...][0], (_N_GRAPHS, _ROW_BLK))
              ).astype(jnp.float32)
    psum = jnp.dot(onehot, h, preferred_element_type=jnp.float32)
    pcnt = jnp.dot(onehot, jnp.ones((_ROW_BLK, _NQ), jnp.float32),
                   preferred_element_type=jnp.float32)

    @pl.when(i == 0)
    def _():
        sums[...] = jnp.zeros_like(sums)
        cnts[...] = jnp.zeros_like(cnts)

    sums[...] += psum
    cnts[...] += pcnt

    @pl.when(i == (_N_NODES // _ROW_BLK) - 1)
    def _():
        pooled = sums[...] / jnp.maximum(cnts[...], 1.0)
        o_ref[...] = (jnp.dot(pooled, wc_ref[...],
                              preferred_element_type=jnp.float32)
                      + bc_ref[...])


def kernel(x, edge_index, batch, W0, theta0, W1, theta1, Wc, bc):
    f32 = jnp.float32
    nblk = _N_NODES // _ROW_BLK

    src = edge_index[0]
    dst = edge_index[1]
    pad = _E_PAD - _N_EDGES
    src_w = jnp.concatenate(
        [src, jnp.zeros((pad,), jnp.int32)]).reshape(_NW, _GROUPS, _GSZ)
    dst_w = jnp.concatenate(
        [dst, jnp.full((pad,), _TRASH, jnp.int32)]
    ).reshape(_NW, _GROUPS * _CH_PER_G, _CH)
    z2 = jnp.zeros((_RPT, _NQ), f32)
    z1 = jnp.zeros((_RPT,), f32)
    ones128 = jnp.ones((_CH,), f32)
    c0, s0 = jnp.cos(theta0), jnp.sin(theta0)
    c1, s1 = jnp.cos(theta1), jnp.sin(theta1)
    batch_w = batch.reshape(nblk, 1, _ROW_BLK)
    bc2 = bc.reshape(1, _OUT)

    y0 = pl.pallas_call(
        _mm0_body,
        grid=(nblk,),
        in_specs=[pl.BlockSpec((_ROW_BLK, _D_FEAT), lambda i: (i, 0)),
                  pl.BlockSpec((_D_FEAT, _NQ), lambda i: (0, 0))],
        out_specs=pl.BlockSpec((_ROW_BLK, _NQ), lambda i: (i, 0)),
        out_shape=jax.ShapeDtypeStruct((_N_NODES, _NQ), f32),
    )(x, W0)

    agg0, deg0 = _sc_agg(y0, src_w, dst_w, z2, z1, ones128)
    deg0r = deg0.reshape(2, _NROWS, 1)

    y1 = pl.pallas_call(
        _mid_body,
        grid=(nblk,),
        in_specs=[pl.BlockSpec((2, _ROW_BLK, _NQ), lambda i: (0, i, 0)),
                  pl.BlockSpec((2, _ROW_BLK, 1), lambda i: (0, i, 0)),
                  pl.BlockSpec((2, _NQ), lambda i: (0, 0)),
                  pl.BlockSpec((2, _NQ), lambda i: (0, 0)),
                  pl.BlockSpec((_NQ, _NQ), lambda i: (0, 0))],
        out_specs=pl.BlockSpec((_ROW_BLK, _NQ), lambda i: (i, 0)),
        out_shape=jax.ShapeDtypeStruct((_N_NODES, _NQ), f32),
    )(agg0, deg0r, c0, s0, W1)

    agg1, deg1 = _sc_agg(y1, src_w, dst_w, z2, z1, ones128)
    deg1r = deg1.reshape(2, _NROWS, 1)

    out = pl.pallas_call(
        _fin_body,
        grid=(nblk,),
        in_specs=[pl.BlockSpec((2, _ROW_BLK, _NQ), lambda i: (0, i, 0)),
                  pl.BlockSpec((2, _ROW_BLK, 1), lambda i: (0, i, 0)),
                  pl.BlockSpec((2, _NQ), lambda i: (0, 0)),
                  pl.BlockSpec((2, _NQ), lambda i: (0, 0)),
                  pl.BlockSpec((1, 1, _ROW_BLK), lambda i: (i, 0, 0)),
                  pl.BlockSpec((_NQ, _OUT), lambda i: (0, 0)),
                  pl.BlockSpec((1, _OUT), lambda i: (0, 0))],
        out_specs=pl.BlockSpec((_N_GRAPHS, _OUT), lambda i: (0, 0)),
        out_shape=jax.ShapeDtypeStruct((_N_GRAPHS, _OUT), f32),
        scratch_shapes=[pltpu.VMEM((_N_GRAPHS, _NQ), f32),
                        pltpu.VMEM((_N_GRAPHS, _NQ), f32)],
    )(agg1, deg1r, c1, s1, batch_w, Wc, bc2)

    return out


# speedup vs baseline: 16.6164x; 1.2928x over previous
import functools

import jax
import jax.numpy as jnp
from jax import lax
from jax.experimental import pallas as pl
from jax.experimental.pallas import tpu as pltpu
from jax.experimental.pallas import tpu_sc as plsc

_N_NODES = 10000
_N_EDGES = 320000
_D_FEAT = 128
_NQ = 16
_N_GRAPHS = 64
_OUT = 10

_NW = 32
_EDGES_PER_W = 10240
_E_PAD = _NW * _EDGES_PER_W
_GROUPS = 10
_GSZ = 1024
_CH = 128
_CH_PER_G = _GSZ // _CH
_NROWS = 10240
_TRASH = _N_NODES
_RPT = _NROWS // 16

_ROW_BLK = 1000


def _make_sc_agg(with_deg):
    def body(y_hbm, src_hbm, dst_hbm, z2_hbm, z1_hbm, ones_hbm,
             *out_and_scratch):
        if with_deg:
            (agg_out, deg_out, src_v, dst_v, rows_v, ones_v, acc_s, deg_s,
             gsem, ssem) = out_and_scratch
        else:
            (agg_out, src_v, dst_v, rows_v, acc_s, gsem, ssem) = \
                out_and_scratch
        cid = lax.axis_index("c")
        sid = lax.axis_index("s")
        wid = sid * 2 + cid

        pltpu.sync_copy(z2_hbm, acc_s.at[pl.ds(sid * _RPT, _RPT)])
        if with_deg:
            pltpu.sync_copy(z1_hbm, deg_s.at[pl.ds(sid * _RPT, _RPT)])
            pltpu.sync_copy(ones_hbm, ones_v)
        pltpu.sync_copy(src_hbm.at[wid], src_v)
        pltpu.sync_copy(dst_hbm.at[wid], dst_v)
        plsc.subcore_barrier()

        pltpu.async_copy(y_hbm.at[src_v.at[0]], rows_v.at[0], gsem)

        def group(g, carry):
            buf = lax.rem(g, 2)
            rows = rows_v.at[buf]
            pltpu.make_async_copy(y_hbm.at[src_v.at[g]], rows, gsem).wait()

            @pl.when(g + 1 < _GROUPS)
            def _():
                pltpu.async_copy(y_hbm.at[src_v.at[g + 1]],
                                 rows_v.at[1 - buf], gsem)

            for j in range(_CH_PER_G):
                idx = dst_v.at[g * _CH_PER_G + j]
                pltpu.async_copy(rows.at[pl.ds(j * _CH, _CH)], acc_s.at[idx],
                                 ssem, add=True)
                if with_deg:
                    pltpu.async_copy(ones_v, deg_s.at[idx], ssem, add=True)
            for j in range(_CH_PER_G):
                idx = dst_v.at[g * _CH_PER_G + j]
                pltpu.make_async_copy(rows.at[pl.ds(j * _CH, _CH)],
                                      acc_s.at[idx], ssem).wait()
                if with_deg:
                    pltpu.make_async_copy(ones_v, deg_s.at[idx], ssem).wait()
            return carry

        lax.fori_loop(0, _GROUPS, group, 0)
        plsc.subcore_barrier()

        pltpu.sync_copy(acc_s.at[pl.ds(sid * _RPT, _RPT)],
                        agg_out.at[cid, pl.ds(sid * _RPT, _RPT)])
        if with_deg:
            pltpu.sync_copy(deg_s.at[pl.ds(sid * _RPT, _RPT)],
                            deg_out.at[cid, pl.ds(sid * _RPT, _RPT)])

    out_type = [jax.ShapeDtypeStruct((2, _NROWS, _NQ), jnp.float32)]
    scratch = [
        pltpu.VMEM((_GROUPS, _GSZ), jnp.int32),
        pltpu.VMEM((_GROUPS * _CH_PER_G, _CH), jnp.int32),
        pltpu.VMEM((2, _GSZ, _NQ), jnp.float32),
        pltpu.VMEM_SHARED((_NROWS, _NQ), jnp.float32),
        pltpu.SemaphoreType.DMA,
        pltpu.SemaphoreType.DMA,
    ]
    if with_deg:
        out_type.append(jax.ShapeDtypeStruct((2, _NROWS), jnp.float32))
        scratch.insert(3, pltpu.VMEM((_CH,), jnp.float32))
        scratch.insert(5, pltpu.VMEM_SHARED((_NROWS,), jnp.float32))
    return pl.kernel(
        body,
        out_type=tuple(out_type),
        mesh=plsc.VectorSubcoreMesh(core_axis_name="c", subcore_axis_name="s"),
        scratch_types=scratch,
        compiler_params=pltpu.CompilerParams(use_tc_tiling_on_sc=False),
    )


_sc_agg_deg = _make_sc_agg(True)
_sc_agg_nodeg = _make_sc_agg(False)


def _mm0_body(x_ref, w_ref, o_ref):
    o_ref[...] = jnp.dot(x_ref[...], w_ref[...],
                         preferred_element_type=jnp.float32)


def _shift_mat():
    row = lax.broadcasted_iota(jnp.int32, (_NQ, _NQ), 0)
    col = lax.broadcasted_iota(jnp.int32, (_NQ, _NQ), 1)
    return (col == (row + 1) % _NQ).astype(jnp.float32)


def _post(aggp, degp, c, s):
    a = aggp[0] + aggp[1]
    d = degp[0] + degp[1]
    h = jnp.tanh(a / jnp.maximum(d, 1.0))
    S = _shift_mat()
    for dd in range(2):
        h = (c[dd][None, :] * h
             + s[dd][None, :] * jnp.dot(h, S, preferred_element_type=jnp.float32))
    return jnp.where(h >= 0, h, 0.2 * h)


def _mid_body(aggp_ref, degp_ref, c_ref, s_ref, w_ref, o_ref):
    h = _post(aggp_ref[...], degp_ref[...], c_ref[...], s_ref[...])
    o_ref[...] = jnp.dot(h, w_ref[...], preferred_element_type=jnp.float32)


def _fin_body(aggp_ref, degp_ref, c_ref, s_ref, b_ref, wc_ref, bc_ref, o_ref,
              sums, cnts):
    i = pl.program_id(0)
    h = _post(aggp_ref[...], degp_ref[...], c_ref[...], s_ref[...])
    onehot = (lax.broadcasted_iota(jnp.int32, (_N_GRAPHS, _ROW_BLK), 0)
              == jnp.broadcast_to(b_ref[...][0], (_N_GRAPHS, _ROW_BLK))
              ).astype(jnp.float32)
    psum = jnp.dot(onehot, h, preferred_element_type=jnp.float32)
    pcnt = jnp.dot(onehot, jnp.ones((_ROW_BLK, _NQ), jnp.float32),
                   preferred_element_type=jnp.float32)

    @pl.when(i == 0)
    def _():
        sums[...] = jnp.zeros_like(sums)
        cnts[...] = jnp.zeros_like(cnts)

    sums[...] += psum
    cnts[...] += pcnt

    @pl.when(i == (_N_NODES // _ROW_BLK) - 1)
    def _():
        pooled = sums[...] / jnp.maximum(cnts[...], 1.0)
        o_ref[...] = (jnp.dot(pooled, wc_ref[...],
                              preferred_element_type=jnp.float32)
                      + bc_ref[...])


def kernel(x, edge_index, batch, W0, theta0, W1, theta1, Wc, bc):
    f32 = jnp.float32
    nblk = _N_NODES // _ROW_BLK

    src = edge_index[0]
    dst = edge_index[1]
    pad = _E_PAD - _N_EDGES
    src_w = jnp.concatenate(
        [src, jnp.zeros((pad,), jnp.int32)]).reshape(_NW, _GROUPS, _GSZ)
    dst_w = jnp.concatenate(
        [dst, jnp.full((pad,), _TRASH, jnp.int32)]
    ).reshape(_NW, _GROUPS * _CH_PER_G, _CH)
    z2 = jnp.zeros((_RPT, _NQ), f32)
    z1 = jnp.zeros((_RPT,), f32)
    ones128 = jnp.ones((_CH,), f32)
    c0, s0 = jnp.cos(theta0), jnp.sin(theta0)
    c1, s1 = jnp.cos(theta1), jnp.sin(theta1)
    batch_w = batch.reshape(nblk, 1, _ROW_BLK)
    bc2 = bc.reshape(1, _OUT)

    y0 = pl.pallas_call(
        _mm0_body,
        grid=(nblk,),
        in_specs=[pl.BlockSpec((_ROW_BLK, _D_FEAT), lambda i: (i, 0)),
                  pl.BlockSpec((_D_FEAT, _NQ), lambda i: (0, 0))],
        out_specs=pl.BlockSpec((_ROW_BLK, _NQ), lambda i: (i, 0)),
        out_shape=jax.ShapeDtypeStruct((_N_NODES, _NQ), f32),
    )(x, W0)

    agg0, deg0 = _sc_agg_deg(y0, src_w, dst_w, z2, z1, ones128)
    deg0r = deg0.reshape(2, _NROWS, 1)

    y1 = pl.pallas_call(
        _mid_body,
        grid=(nblk,),
        in_specs=[pl.BlockSpec((2, _ROW_BLK, _NQ), lambda i: (0, i, 0)),
                  pl.BlockSpec((2, _ROW_BLK, 1), lambda i: (0, i, 0)),
                  pl.BlockSpec((2, _NQ), lambda i: (0, 0)),
                  pl.BlockSpec((2, _NQ), lambda i: (0, 0)),
                  pl.BlockSpec((_NQ, _NQ), lambda i: (0, 0))],
        out_specs=pl.BlockSpec((_ROW_BLK, _NQ), lambda i: (i, 0)),
        out_shape=jax.ShapeDtypeStruct((_N_NODES, _NQ), f32),
    )(agg0, deg0r, c0, s0, W1)

    (agg1,) = _sc_agg_nodeg(y1, src_w, dst_w, z2, z1, ones128)

    out = pl.pallas_call(
        _fin_body,
        grid=(nblk,),
        in_specs=[pl.BlockSpec((2, _ROW_BLK, _NQ), lambda i: (0, i, 0)),
                  pl.BlockSpec((2, _ROW_BLK, 1), lambda i: (0, i, 0)),
                  pl.BlockSpec((2, _NQ), lambda i: (0, 0)),
                  pl.BlockSpec((2, _NQ), lambda i: (0, 0)),
                  pl.BlockSpec((1, 1, _ROW_BLK), lambda i: (i, 0, 0)),
                  pl.BlockSpec((_NQ, _OUT), lambda i: (0, 0)),
                  pl.BlockSpec((1, _OUT), lambda i: (0, 0))],
        out_specs=pl.BlockSpec((_N_GRAPHS, _OUT), lambda i: (0, 0)),
        out_shape=jax.ShapeDtypeStruct((_N_GRAPHS, _OUT), f32),
        scratch_shapes=[pltpu.VMEM((_N_GRAPHS, _NQ), f32),
                        pltpu.VMEM((_N_GRAPHS, _NQ), f32)],
    )(agg1, deg0r, c1, s1, batch_w, Wc, bc2)

    return out


# speedup vs baseline: 23.0136x; 1.3850x over previous
import functools

import jax
import jax.numpy as jnp
from jax import lax
from jax.experimental import pallas as pl
from jax.experimental.pallas import tpu as pltpu
from jax.experimental.pallas import tpu_sc as plsc

_N_NODES = 10000
_N_EDGES = 320000
_D_FEAT = 128
_NQ = 16
_N_GRAPHS = 64
_OUT = 10

_NW = 32
_EDGES_PER_W = 10240
_E_PAD = _NW * _EDGES_PER_W
_GROUPS = 10
_GSZ = 1024
_CH = 128
_CH_PER_G = _GSZ // _CH
_NROWS = 10240
_TRASH = _N_NODES
_RPT = _NROWS // 16

_ROW_BLK = 1000
_YPT = _N_NODES // 16


def _make_sc_agg(with_deg):
    def body(y_hbm, src_hbm, dst_hbm, z2_hbm, z1_hbm, ones_hbm,
             *out_and_scratch):
        if with_deg:
            (agg_out, deg_out, src_v, dst_v, rows_v, ones_v, acc_s, deg_s,
             y_s, gsem, ssem) = out_and_scratch
        else:
            (agg_out, src_v, dst_v, rows_v, acc_s, y_s, gsem, ssem) = \
                out_and_scratch
        cid = lax.axis_index("c")
        sid = lax.axis_index("s")
        wid = sid * 2 + cid

        pltpu.sync_copy(z2_hbm, acc_s.at[pl.ds(sid * _RPT, _RPT)])
        pltpu.sync_copy(y_hbm.at[pl.ds(sid * _YPT, _YPT)],
                        y_s.at[pl.ds(sid * _YPT, _YPT)])
        if with_deg:
            pltpu.sync_copy(z1_hbm, deg_s.at[pl.ds(sid * _RPT, _RPT)])
            pltpu.sync_copy(ones_hbm, ones_v)
        pltpu.sync_copy(src_hbm.at[wid], src_v)
        pltpu.sync_copy(dst_hbm.at[wid], dst_v)
        plsc.subcore_barrier()

        pltpu.async_copy(y_s.at[src_v.at[0]], rows_v.at[0], gsem)

        def group(g, carry):
            buf = lax.rem(g, 2)
            rows = rows_v.at[buf]
            pltpu.make_async_copy(y_s.at[src_v.at[g]], rows, gsem).wait()

            @pl.when(g + 1 < _GROUPS)
            def _():
                pltpu.async_copy(y_s.at[src_v.at[g + 1]],
                                 rows_v.at[1 - buf], gsem)

            for j in range(_CH_PER_G):
                idx = dst_v.at[g * _CH_PER_G + j]
                pltpu.async_copy(rows.at[pl.ds(j * _CH, _CH)], acc_s.at[idx],
                                 ssem, add=True)
                if with_deg:
                    pltpu.async_copy(ones_v, deg_s.at[idx], ssem, add=True)
            for j in range(_CH_PER_G):
                idx = dst_v.at[g * _CH_PER_G + j]
                pltpu.make_async_copy(rows.at[pl.ds(j * _CH, _CH)],
                                      acc_s.at[idx], ssem).wait()
                if with_deg:
                    pltpu.make_async_copy(ones_v, deg_s.at[idx], ssem).wait()
            return carry

        lax.fori_loop(0, _GROUPS, group, 0)
        plsc.subcore_barrier()

        pltpu.sync_copy(acc_s.at[pl.ds(sid * _RPT, _RPT)],
                        agg_out.at[cid, pl.ds(sid * _RPT, _RPT)])
        if with_deg:
            pltpu.sync_copy(deg_s.at[pl.ds(sid * _RPT, _RPT)],
                            deg_out.at[cid, pl.ds(sid * _RPT, _RPT)])

    out_type = [jax.ShapeDtypeStruct((2, _NROWS, _NQ), jnp.float32)]
    scratch = [
        pltpu.VMEM((_GROUPS, _GSZ), jnp.int32),
        pltpu.VMEM((_GROUPS * _CH_PER_G, _CH), jnp.int32),
        pltpu.VMEM((2, _GSZ, _NQ), jnp.float32),
        pltpu.VMEM_SHARED((_NROWS, _NQ), jnp.float32),
        pltpu.VMEM_SHARED((_N_NODES, _NQ), jnp.float32),
        pltpu.SemaphoreType.DMA,
        pltpu.SemaphoreType.DMA,
    ]
    if with_deg:
        out_type.append(jax.ShapeDtypeStruct((2, _NROWS), jnp.float32))
        scratch.insert(3, pltpu.VMEM((_CH,), jnp.float32))
        scratch.insert(5, pltpu.VMEM_SHARED((_NROWS,), jnp.float32))
    return pl.kernel(
        body,
        out_type=tuple(out_type),
        mesh=plsc.VectorSubcoreMesh(core_axis_name="c", subcore_axis_name="s"),
        scratch_types=scratch,
        compiler_params=pltpu.CompilerParams(use_tc_tiling_on_sc=False),
    )


_sc_agg_deg = _make_sc_agg(True)
_sc_agg_nodeg = _make_sc_agg(False)


def _mm0_body(x_ref, w_ref, o_ref):
    o_ref[...] = jnp.dot(x_ref[...], w_ref[...],
                         preferred_element_type=jnp.float32)


def _shift_mat():
    row = lax.broadcasted_iota(jnp.int32, (_NQ, _NQ), 0)
    col = lax.broadcasted_iota(jnp.int32, (_NQ, _NQ), 1)
    return (col == (row + 1) % _NQ).astype(jnp.float32)


def _post(aggp, degp, c, s):
    a = aggp[0] + aggp[1]
    d = degp[0] + degp[1]
    h = jnp.tanh(a / jnp.maximum(d, 1.0))
    S = _shift_mat()
    for dd in range(2):
        h = (c[dd][None, :] * h
             + s[dd][None, :] * jnp.dot(h, S, preferred_element_type=jnp.float32))
    return jnp.where(h >= 0, h, 0.2 * h)


def _mid_body(aggp_ref, degp_ref, c_ref, s_ref, w_ref, o_ref):
    h = _post(aggp_ref[...], degp_ref[...], c_ref[...], s_ref[...])
    o_ref[...] = jnp.dot(h, w_ref[...], preferred_element_type=jnp.float32)


def _fin_body(aggp_ref, degp_ref, c_ref, s_ref, b_ref, wc_ref, bc_ref, o_ref,
              sums, cnts):
    i = pl.program_id(0)
    h = _post(aggp_ref[...], degp_ref[...], c_ref[...], s_ref[...])
    onehot = (lax.broadcasted_iota(jnp.int32, (_N_GRAPHS, _ROW_BLK), 0)
              == jnp.broadcast_to(b_ref[...][0], (_N_GRAPHS, _ROW_BLK))
              ).astype(jnp.float32)
    psum = jnp.dot(onehot, h, preferred_element_type=jnp.float32)
    pcnt = jnp.dot(onehot, jnp.ones((_ROW_BLK, _NQ), jnp.float32),
                   preferred_element_type=jnp.float32)

    @pl.when(i == 0)
    def _():
        sums[...] = jnp.zeros_like(sums)
        cnts[...] = jnp.zeros_like(cnts)

    sums[...] += psum
    cnts[...] += pcnt

    @pl.when(i == (_N_NODES // _ROW_BLK) - 1)
    def _():
        pooled = sums[...] / jnp.maximum(cnts[...], 1.0)
        o_ref[...] = (jnp.dot(pooled, wc_ref[...],
                              preferred_element_type=jnp.float32)
                      + bc_ref[...])


def kernel(x, edge_index, batch, W0, theta0, W1, theta1, Wc, bc):
    f32 = jnp.float32
    nblk = _N_NODES // _ROW_BLK

    src = edge_index[0]
    dst = edge_index[1]
    pad = _E_PAD - _N_EDGES
    src_w = jnp.concatenate(
        [src, jnp.zeros((pad,), jnp.int32)]).reshape(_NW, _GROUPS, _GSZ)
    dst_w = jnp.concatenate(
        [dst, jnp.full((pad,), _TRASH, jnp.int32)]
    ).reshape(_NW, _GROUPS * _CH_PER_G, _CH)
    z2 = jnp.zeros((_RPT, _NQ), f32)
    z1 = jnp.zeros((_RPT,), f32)
    ones128 = jnp.ones((_CH,), f32)
    c0, s0 = jnp.cos(theta0), jnp.sin(theta0)
    c1, s1 = jnp.cos(theta1), jnp.sin(theta1)
    batch_w = batch.reshape(nblk, 1, _ROW_BLK)
    bc2 = bc.reshape(1, _OUT)

    y0 = pl.pallas_call(
        _mm0_body,
        grid=(nblk,),
        in_specs=[pl.BlockSpec((_ROW_BLK, _D_FEAT), lambda i: (i, 0)),
                  pl.BlockSpec((_D_FEAT, _NQ), lambda i: (0, 0))],
        out_specs=pl.BlockSpec((_ROW_BLK, _NQ), lambda i: (i, 0)),
        out_shape=jax.ShapeDtypeStruct((_N_NODES, _NQ), f32),
    )(x, W0)

    agg0, deg0 = _sc_agg_deg(y0, src_w, dst_w, z2, z1, ones128)
    deg0r = deg0.reshape(2, _NROWS, 1)

    y1 = pl.pallas_call(
        _mid_body,
        grid=(nblk,),
        in_specs=[pl.BlockSpec((2, _ROW_BLK, _NQ), lambda i: (0, i, 0)),
                  pl.BlockSpec((2, _ROW_BLK, 1), lambda i: (0, i, 0)),
                  pl.BlockSpec((2, _NQ), lambda i: (0, 0)),
                  pl.BlockSpec((2, _NQ), lambda i: (0, 0)),
                  pl.BlockSpec((_NQ, _NQ), lambda i: (0, 0))],
        out_specs=pl.BlockSpec((_ROW_BLK, _NQ), lambda i: (i, 0)),
        out_shape=jax.ShapeDtypeStruct((_N_NODES, _NQ), f32),
    )(agg0, deg0r, c0, s0, W1)

    (agg1,) = _sc_agg_nodeg(y1, src_w, dst_w, z2, z1, ones128)

    out = pl.pallas_call(
        _fin_body,
        grid=(nblk,),
        in_specs=[pl.BlockSpec((2, _ROW_BLK, _NQ), lambda i: (0, i, 0)),
                  pl.BlockSpec((2, _ROW_BLK, 1), lambda i: (0, i, 0)),
                  pl.BlockSpec((2, _NQ), lambda i: (0, 0)),
                  pl.BlockSpec((2, _NQ), lambda i: (0, 0)),
                  pl.BlockSpec((1, 1, _ROW_BLK), lambda i: (i, 0, 0)),
                  pl.BlockSpec((_NQ, _OUT), lambda i: (0, 0)),
                  pl.BlockSpec((1, _OUT), lambda i: (0, 0))],
        out_specs=pl.BlockSpec((_N_GRAPHS, _OUT), lambda i: (0, 0)),
        out_shape=jax.ShapeDtypeStruct((_N_GRAPHS, _OUT), f32),
        scratch_shapes=[pltpu.VMEM((_N_GRAPHS, _NQ), f32),
                        pltpu.VMEM((_N_GRAPHS, _NQ), f32)],
    )(agg1, deg0r, c1, s1, batch_w, Wc, bc2)

    return out


# speedup vs baseline: 28.2793x; 1.2288x over previous
import jax
import jax.numpy as jnp
from jax import lax
from jax.experimental import pallas as pl
from jax.experimental.pallas import tpu as pltpu
from jax.experimental.pallas import tpu_sc as plsc

_N_NODES = 10000
_N_EDGES = 320000
_D_FEAT = 128
_NQ = 16
_N_GRAPHS = 64
_OUT = 10

_NW = 32
_EDGES_PER_W = 10240
_E_PAD = _NW * _EDGES_PER_W
_GROUPS = 10
_GSZ = 1024
_CH = 128
_CH_PER_G = _GSZ // _CH
_NROWS = 10240
_TRASH = _N_NODES
_RPT = _NROWS // 16
_YPT = _N_NODES // 16

_PACK = 8
_FROWS = _NROWS // _PACK
_FBLK = 128
_NBLK = _FROWS // _FBLK


def _make_sc_agg(with_deg, table_rows):
    def body(y_hbm, src_hbm, dst_hbm, z2_hbm, ones_hbm, *out_and_scratch):
        if with_deg:
            (agg_out, deg_out, src_v, dst_v, rows_v, ones_v, acc_s, deg_s,
             y_s, gsem, ssem) = out_and_scratch
        else:
            (agg_out, src_v, dst_v, rows_v, acc_s, y_s, gsem, ssem) = \
                out_and_scratch
        cid = lax.axis_index("c")
        sid = lax.axis_index("s")
        wid = sid * 2 + cid

        ypt = table_rows // 16
        pltpu.sync_copy(z2_hbm, acc_s.at[pl.ds(sid * _RPT, _RPT)])
        pltpu.sync_copy(y_hbm.at[pl.ds(sid * ypt, ypt)],
                        y_s.at[pl.ds(sid * ypt, ypt)])
        if with_deg:
            pltpu.sync_copy(z2_hbm, deg_s.at[pl.ds(sid * _RPT, _RPT)])
            pltpu.sync_copy(ones_hbm, ones_v)
        pltpu.sync_copy(src_hbm.at[wid], src_v)
        pltpu.sync_copy(dst_hbm.at[wid], dst_v)
        plsc.subcore_barrier()

        pltpu.async_copy(y_s.at[src_v.at[0]], rows_v.at[0], gsem)

        def group(g, carry):
            buf = lax.rem(g, 2)
            rows = rows_v.at[buf]
            pltpu.make_async_copy(y_s.at[src_v.at[g]], rows, gsem).wait()

            @pl.when(g + 1 < _GROUPS)
            def _():
                pltpu.async_copy(y_s.at[src_v.at[g + 1]],
                                 rows_v.at[1 - buf], gsem)

            for j in range(_CH_PER_G):
                idx = dst_v.at[g * _CH_PER_G + j]
                pltpu.async_copy(rows.at[pl.ds(j * _CH, _CH)], acc_s.at[idx],
                                 ssem, add=True)
                if with_deg:
                    pltpu.async_copy(ones_v, deg_s.at[idx], ssem, add=True)
            for j in range(_CH_PER_G):
                idx = dst_v.at[g * _CH_PER_G + j]
                pltpu.make_async_copy(rows.at[pl.ds(j * _CH, _CH)],
                                      acc_s.at[idx], ssem).wait()
                if with_deg:
                    pltpu.make_async_copy(ones_v, deg_s.at[idx], ssem).wait()
            return carry

        lax.fori_loop(0, _GROUPS, group, 0)
        plsc.subcore_barrier()

        pltpu.sync_copy(acc_s.at[pl.ds(sid * _RPT, _RPT)],
                        agg_out.at[cid, pl.ds(sid * _RPT, _RPT)])
        if with_deg:
            pltpu.sync_copy(deg_s.at[pl.ds(sid * _RPT, _RPT)],
                            deg_out.at[cid, pl.ds(sid * _RPT, _RPT)])

    out_type = [jax.ShapeDtypeStruct((2, _NROWS, _NQ), jnp.float32)]
    scratch = [
        pltpu.VMEM((_GROUPS, _GSZ), jnp.int32),
        pltpu.VMEM((_GROUPS * _CH_PER_G, _CH), jnp.int32),
        pltpu.VMEM((2, _GSZ, _NQ), jnp.float32),
        pltpu.VMEM_SHARED((_NROWS, _NQ), jnp.float32),
        pltpu.VMEM_SHARED((table_rows, _NQ), jnp.float32),
        pltpu.SemaphoreType.DMA,
        pltpu.SemaphoreType.DMA,
    ]
    if with_deg:
        out_type.append(jax.ShapeDtypeStruct((2, _NROWS, _NQ), jnp.float32))
        scratch.insert(3, pltpu.VMEM((_CH, _NQ), jnp.float32))
        scratch.insert(5, pltpu.VMEM_SHARED((_NROWS, _NQ), jnp.float32))
    return pl.kernel(
        body,
        out_type=tuple(out_type),
        mesh=plsc.VectorSubcoreMesh(core_axis_name="c", subcore_axis_name="s"),
        scratch_types=scratch,
        compiler_params=pltpu.CompilerParams(use_tc_tiling_on_sc=False),
    )


_sc_agg_deg = _make_sc_agg(True, _N_NODES)
_sc_agg_nodeg = _make_sc_agg(False, _NROWS)


def _mm0_body(x2_ref, w_ref, o_ref):
    o_ref[...] = jnp.dot(x2_ref[...], w_ref[...],
                         preferred_element_type=jnp.float32)


def _shift128():
    row = lax.broadcasted_iota(jnp.int32, (_FBLK, _FBLK), 0)
    col = lax.broadcasted_iota(jnp.int32, (_FBLK, _FBLK), 1)
    same_blk = (row // _NQ) == (col // _NQ)
    rolled = ((row % _NQ) + 1) % _NQ == (col % _NQ)
    return (same_blk & rolled).astype(jnp.float32)


def _post(aggp, degp, c, s):
    a = aggp[0] + aggp[1]
    d = degp[0] + degp[1]
    h = jnp.tanh(a / jnp.maximum(d, 1.0))
    S = _shift128()
    for dd in range(2):
        h = (c[dd][None, :] * h
             + s[dd][None, :] * jnp.dot(h, S,
                                        preferred_element_type=jnp.float32))
    return jnp.where(h >= 0, h, 0.2 * h)


def _mid_body(aggp_ref, degp_ref, c_ref, s_ref, w_ref, o_ref):
    h = _post(aggp_ref[...], degp_ref[...], c_ref[...], s_ref[...])
    o_ref[...] = jnp.dot(h, w_ref[...], preferred_element_type=jnp.float32)


def _fin_body(aggp_ref, degp_ref, c_ref, s_ref, bt_ref, wc_ref, bc_ref,
              o_ref, sums, cnts):
    i = pl.program_id(0)
    h = _post(aggp_ref[...], degp_ref[...], c_ref[...], s_ref[...])
    bt = bt_ref[...]
    giota = lax.broadcasted_iota(jnp.int32, (_N_GRAPHS, _FBLK), 0)
    psum = jnp.zeros((_N_GRAPHS, _NQ), jnp.float32)
    osum = jnp.zeros((_N_GRAPHS, _FBLK), jnp.float32)
    for k in range(_PACK):
        onehot = (giota == bt[k][None, :]).astype(jnp.float32)
        hk = h[:, k * _NQ:(k + 1) * _NQ]
        psum += jnp.dot(onehot, hk, preferred_element_type=jnp.float32)
        osum += onehot
    pcnt = jnp.dot(osum, jnp.ones((_FBLK, _NQ), jnp.float32),
                   preferred_element_type=jnp.float32)

    @pl.when(i == 0)
    def _():
        sums[...] = jnp.zeros_like(sums)
        cnts[...] = jnp.zeros_like(cnts)

    sums[...] += psum
    cnts[...] += pcnt

    @pl.when(i == _NBLK - 1)
    def _():
        pooled = sums[...] / jnp.maximum(cnts[...], 1.0)
        o_ref[...] = (jnp.dot(pooled, wc_ref[...],
                              preferred_element_type=jnp.float32)
                      + bc_ref[...])


def _blockdiag(w):
    return jnp.kron(jnp.eye(_PACK, dtype=jnp.float32), w)


def kernel(x, edge_index, batch, W0, theta0, W1, theta1, Wc, bc):
    f32 = jnp.float32

    src = edge_index[0]
    dst = edge_index[1]
    pad = _E_PAD - _N_EDGES
    src_w = jnp.concatenate(
        [src, jnp.zeros((pad,), jnp.int32)]).reshape(_NW, _GROUPS, _GSZ)
    dst_w = jnp.concatenate(
        [dst, jnp.full((pad,), _TRASH, jnp.int32)]
    ).reshape(_NW, _GROUPS * _CH_PER_G, _CH)
    z2 = jnp.zeros((_RPT, _NQ), f32)
    ones16 = jnp.ones((_CH, _NQ), f32)
    c0t = jnp.tile(jnp.cos(theta0), (1, _PACK))
    s0t = jnp.tile(jnp.sin(theta0), (1, _PACK))
    c1t = jnp.tile(jnp.cos(theta1), (1, _PACK))
    s1t = jnp.tile(jnp.sin(theta1), (1, _PACK))
    W0big = jnp.kron(jnp.eye(_PACK, dtype=f32), W0)
    W1big = _blockdiag(W1)
    batch_pad = jnp.concatenate(
        [batch, jnp.full((_NROWS - _N_NODES,), _N_GRAPHS, jnp.int32)])
    batchT = batch_pad.reshape(_FROWS, _PACK).T
    bc2 = bc.reshape(1, _OUT)
    x2 = x.reshape(_N_NODES // _PACK, _PACK * _D_FEAT)

    y0f = pl.pallas_call(
        _mm0_body,
        in_specs=[pl.BlockSpec(x2.shape, lambda: (0, 0)),
                  pl.BlockSpec(W0big.shape, lambda: (0, 0))],
        out_specs=pl.BlockSpec((_N_NODES // _PACK, _FBLK), lambda: (0, 0)),
        out_shape=jax.ShapeDtypeStruct((_N_NODES // _PACK, _FBLK), f32),
    )(x2, W0big)

    agg0, deg0 = _sc_agg_deg(y0f.reshape(_N_NODES, _NQ), src_w, dst_w,
                             z2, ones16)
    agg0f = agg0.reshape(2, _FROWS, _FBLK)
    deg0f = deg0.reshape(2, _FROWS, _FBLK)

    y1f = pl.pallas_call(
        _mid_body,
        grid=(_NBLK,),
        in_specs=[pl.BlockSpec((2, _FBLK, _FBLK), lambda i: (0, i, 0)),
                  pl.BlockSpec((2, _FBLK, _FBLK), lambda i: (0, i, 0)),
                  pl.BlockSpec((2, _FBLK), lambda i: (0, 0)),
                  pl.BlockSpec((2, _FBLK), lambda i: (0, 0)),
                  pl.BlockSpec((_FBLK, _FBLK), lambda i: (0, 0))],
        out_specs=pl.BlockSpec((_FBLK, _FBLK), lambda i: (i, 0)),
        out_shape=jax.ShapeDtypeStruct((_FROWS, _FBLK), f32),
    )(agg0f, deg0f, c0t, s0t, W1big)

    (agg1,) = _sc_agg_nodeg(y1f.reshape(_NROWS, _NQ), src_w, dst_w,
                            z2, ones16)
    agg1f = agg1.reshape(2, _FROWS, _FBLK)

    out = pl.pallas_call(
        _fin_body,
        grid=(_NBLK,),
        in_specs=[pl.BlockSpec((2, _FBLK, _FBLK), lambda i: (0, i, 0)),
                  pl.BlockSpec((2, _FBLK, _FBLK), lambda i: (0, i, 0)),
                  pl.BlockSpec((2, _FBLK), lambda i: (0, 0)),
                  pl.BlockSpec((2, _FBLK), lambda i: (0, 0)),
                  pl.BlockSpec((_PACK, _FBLK), lambda i: (0, i)),
                  pl.BlockSpec((_NQ, _OUT), lambda i: (0, 0)),
                  pl.BlockSpec((1, _OUT), lambda i: (0, 0))],
        out_specs=pl.BlockSpec((_N_GRAPHS, _OUT), lambda i: (0, 0)),
        out_shape=jax.ShapeDtypeStruct((_N_GRAPHS, _OUT), f32),
        scratch_shapes=[pltpu.VMEM((_N_GRAPHS, _NQ), f32),
                        pltpu.VMEM((_N_GRAPHS, _NQ), f32)],
    )(agg1f, deg0f, c1t, s1t, batchT, Wc, bc2)

    return out


# speedup vs baseline: 33.9313x; 1.1999x over previous
import jax
import jax.numpy as jnp
from jax import lax
from jax.experimental import pallas as pl
from jax.experimental.pallas import tpu as pltpu
from jax.experimental.pallas import tpu_sc as plsc

_N_NODES = 10000
_N_EDGES = 320000
_D_FEAT = 128
_NQ = 16
_N_GRAPHS = 64
_OUT = 10

_NW = 32
_EPW = _N_EDGES // _NW
_GSZ = 1024
_NFULL = _EPW // _GSZ
_EPI = _EPW - _NFULL * _GSZ
_CH = 128
_CH_PER_G = _GSZ // _CH
_EPI_CH = _EPI // _CH
_TAIL = _EPI - _EPI_CH * _CH
_NROWS = 10240
_RPT = _NROWS // 16

_PACK = 8
_FROWS = _NROWS // _PACK
_FBLK = 128
_NBLK = _FROWS // _FBLK


def _make_sc_agg(with_deg, table_rows):
    def body(y_hbm, ei_hbm, z2_hbm, ones_hbm, *out_and_scratch):
        if with_deg:
            (agg_out, deg_out, src_v, dst_v, rows_v, ones_v, acc_s, deg_s,
             y_s, gsem, ssem) = out_and_scratch
        else:
            (agg_out, src_v, dst_v, rows_v, acc_s, y_s, gsem, ssem) = \
                out_and_scratch
        cid = lax.axis_index("c")
        sid = lax.axis_index("s")
        wid = sid * 2 + cid
        base = wid * _EPW

        ypt = table_rows // 16
        pltpu.sync_copy(z2_hbm, acc_s.at[pl.ds(sid * _RPT, _RPT)])
        pltpu.sync_copy(y_hbm.at[pl.ds(sid * ypt, ypt)],
                        y_s.at[pl.ds(sid * ypt, ypt)])
        if with_deg:
            pltpu.sync_copy(z2_hbm, deg_s.at[pl.ds(sid * _RPT, _RPT)])
            pltpu.sync_copy(ones_hbm, ones_v)
        pltpu.sync_copy(ei_hbm.at[0, pl.ds(base, _EPW)], src_v)
        pltpu.sync_copy(ei_hbm.at[1, pl.ds(base, _EPW)], dst_v)
        plsc.subcore_barrier()

        def scat(eoff, rows, roff, n):
            idx = dst_v.at[pl.ds(eoff, n)]
            pltpu.async_copy(rows.at[pl.ds(roff, n)], acc_s.at[idx],
                             ssem, add=True)
            if with_deg:
                pltpu.async_copy(ones_v.at[pl.ds(0, n)], deg_s.at[idx],
                                 ssem, add=True)

        def scat_wait(eoff, rows, roff, n):
            idx = dst_v.at[pl.ds(eoff, n)]
            pltpu.make_async_copy(rows.at[pl.ds(roff, n)], acc_s.at[idx],
                                  ssem).wait()
            if with_deg:
                pltpu.make_async_copy(ones_v.at[pl.ds(0, n)], deg_s.at[idx],
                                      ssem).wait()

        pltpu.async_copy(y_s.at[src_v.at[pl.ds(0, _GSZ)]], rows_v.at[0],
                         gsem)

        def group(g, carry):
            buf = lax.rem(g, 2)
            rows = rows_v.at[buf]
            pltpu.make_async_copy(y_s.at[src_v.at[pl.ds(0, _GSZ)]], rows,
                                  gsem).wait()

            @pl.when(g + 1 < _NFULL)
            def _():
                pltpu.async_copy(
                    y_s.at[src_v.at[pl.ds((g + 1) * _GSZ, _GSZ)]],
                    rows_v.at[1 - buf], gsem)

            @pl.when(g + 1 == _NFULL)
            def _():
                pltpu.async_copy(
                    y_s.at[src_v.at[pl.ds(_NFULL * _GSZ, _EPI)]],
                    rows_v.at[1 - buf].at[pl.ds(0, _EPI)], gsem)

            for j in range(_CH_PER_G):
                scat(g * _GSZ + j * _CH, rows, j * _CH, _CH)
            for j in range(_CH_PER_G):
                scat_wait(g * _GSZ + j * _CH, rows, j * _CH, _CH)
            return carry

        lax.fori_loop(0, _NFULL, group, 0)
        erows = rows_v.at[_NFULL % 2]
        pltpu.make_async_copy(y_s.at[src_v.at[pl.ds(0, _EPI)]],
                              erows.at[pl.ds(0, _EPI)], gsem).wait()
        ebase = _NFULL * _GSZ
        for j in range(_EPI_CH):
            scat(ebase + j * _CH, erows, j * _CH, _CH)
        scat(ebase + _EPI_CH * _CH, erows, _EPI_CH * _CH, _TAIL)
        for j in range(_EPI_CH):
            scat_wait(ebase + j * _CH, erows, j * _CH, _CH)
        scat_wait(ebase + _EPI_CH * _CH, erows, _EPI_CH * _CH, _TAIL)
        plsc.subcore_barrier()

        pltpu.sync_copy(acc_s.at[pl.ds(sid * _RPT, _RPT)],
                        agg_out.at[cid, pl.ds(sid * _RPT, _RPT)])
        if with_deg:
            pltpu.sync_copy(deg_s.at[pl.ds(sid * _RPT, _RPT)],
                            deg_out.at[cid, pl.ds(sid * _RPT, _RPT)])

    out_type = [jax.ShapeDtypeStruct((2, _NROWS, _NQ), jnp.float32)]
    scratch = [
        pltpu.VMEM((_EPW,), jnp.int32),
        pltpu.VMEM((_EPW,), jnp.int32),
        pltpu.VMEM((2, _GSZ, _NQ), jnp.float32),
        pltpu.VMEM_SHARED((_NROWS, _NQ), jnp.float32),
        pltpu.VMEM_SHARED((table_rows, _NQ), jnp.float32),
        pltpu.SemaphoreType.DMA,
        pltpu.SemaphoreType.DMA,
    ]
    if with_deg:
        out_type.append(jax.ShapeDtypeStruct((2, _NROWS, _NQ), jnp.float32))
        scratch.insert(3, pltpu.VMEM((_CH, _NQ), jnp.float32))
        scratch.insert(5, pltpu.VMEM_SHARED((_NROWS, _NQ), jnp.float32))
    return pl.kernel(
        body,
        out_type=tuple(out_type),
        mesh=plsc.VectorSubcoreMesh(core_axis_name="c", subcore_axis_name="s"),
        scratch_types=scratch,
        compiler_params=pltpu.CompilerParams(use_tc_tiling_on_sc=False),
    )


_sc_agg_deg = _make_sc_agg(True, _N_NODES)
_sc_agg_nodeg = _make_sc_agg(False, _NROWS)


def _mm0_body(x2_ref, w_ref, o_ref):
    o_ref[...] = jnp.dot(x2_ref[...], w_ref[...],
                         preferred_element_type=jnp.float32)


def _shift128():
    row = lax.broadcasted_iota(jnp.int32, (_FBLK, _FBLK), 0)
    col = lax.broadcasted_iota(jnp.int32, (_FBLK, _FBLK), 1)
    same_blk = (row // _NQ) == (col // _NQ)
    rolled = ((row % _NQ) + 1) % _NQ == (col % _NQ)
    return (same_blk & rolled).astype(jnp.float32)


def _post(aggp, degp, c, s):
    a = aggp[0] + aggp[1]
    d = degp[0] + degp[1]
    h = jnp.tanh(a / jnp.maximum(d, 1.0))
    S = _shift128()
    for dd in range(2):
        h = (c[dd][None, :] * h
             + s[dd][None, :] * jnp.dot(h, S,
                                        preferred_element_type=jnp.float32))
    return jnp.where(h >= 0, h, 0.2 * h)


def _mid_body(aggp_ref, degp_ref, c_ref, s_ref, w_ref, o_ref):
    h = _post(aggp_ref[...], degp_ref[...], c_ref[...], s_ref[...])
    o_ref[...] = jnp.dot(h, w_ref[...], preferred_element_type=jnp.float32)


def _fin_body(aggp_ref, degp_ref, c_ref, s_ref, bt_ref, wc_ref, bc_ref,
              o_ref, sums, cnts):
    i = pl.program_id(0)
    h = _post(aggp_ref[...], degp_ref[...], c_ref[...], s_ref[...])
    bt = bt_ref[...]
    giota = lax.broadcasted_iota(jnp.int32, (_N_GRAPHS, _FBLK), 0)
    psum = jnp.zeros((_N_GRAPHS, _NQ), jnp.float32)
    osum = jnp.zeros((_N_GRAPHS, _FBLK), jnp.float32)
    for k in range(_PACK):
        onehot = (giota == bt[k][None, :]).astype(jnp.float32)
        hk = h[:, k * _NQ:(k + 1) * _NQ]
        psum += jnp.dot(onehot, hk, preferred_element_type=jnp.float32)
        osum += onehot
    pcnt = jnp.dot(osum, jnp.ones((_FBLK, _NQ), jnp.float32),
                   preferred_element_type=jnp.float32)

    @pl.when(i == 0)
    def _():
        sums[...] = jnp.zeros_like(sums)
        cnts[...] = jnp.zeros_like(cnts)

    sums[...] += psum
    cnts[...] += pcnt

    @pl.when(i == _NBLK - 1)
    def _():
        pooled = sums[...] / jnp.maximum(cnts[...], 1.0)
        o_ref[...] = (jnp.dot(pooled, wc_ref[...],
                              preferred_element_type=jnp.float32)
                      + bc_ref[...])


def _blockdiag(w):
    return jnp.kron(jnp.eye(_PACK, dtype=jnp.float32), w)


def kernel(x, edge_index, batch, W0, theta0, W1, theta1, Wc, bc):
    f32 = jnp.float32

    z2 = jnp.zeros((_RPT, _NQ), f32)
    ones16 = jnp.ones((_CH, _NQ), f32)
    c0t = jnp.tile(jnp.cos(theta0), (1, _PACK))
    s0t = jnp.tile(jnp.sin(theta0), (1, _PACK))
    c1t = jnp.tile(jnp.cos(theta1), (1, _PACK))
    s1t = jnp.tile(jnp.sin(theta1), (1, _PACK))
    W0big = jnp.kron(jnp.eye(_PACK, dtype=f32), W0)
    W1big = _blockdiag(W1)
    batch_pad = jnp.concatenate(
        [batch, jnp.full((_NROWS - _N_NODES,), _N_GRAPHS, jnp.int32)])
    batchT = batch_pad.reshape(_FROWS, _PACK).T
    bc2 = bc.reshape(1, _OUT)
    x2 = x.reshape(_N_NODES // _PACK, _PACK * _D_FEAT)

    y0f = pl.pallas_call(
        _mm0_body,
        in_specs=[pl.BlockSpec(x2.shape, lambda: (0, 0)),
                  pl.BlockSpec(W0big.shape, lambda: (0, 0))],
        out_specs=pl.BlockSpec((_N_NODES // _PACK, _FBLK), lambda: (0, 0)),
        out_shape=jax.ShapeDtypeStruct((_N_NODES // _PACK, _FBLK), f32),
    )(x2, W0big)

    agg0, deg0 = _sc_agg_deg(y0f.reshape(_N_NODES, _NQ), edge_index,
                             z2, ones16)
    agg0f = agg0.reshape(2, _FROWS, _FBLK)
    deg0f = deg0.reshape(2, _FROWS, _FBLK)

    y1f = pl.pallas_call(
        _mid_body,
        grid=(_NBLK,),
        in_specs=[pl.BlockSpec((2, _FBLK, _FBLK), lambda i: (0, i, 0)),
                  pl.BlockSpec((2, _FBLK, _FBLK), lambda i: (0, i, 0)),
                  pl.BlockSpec((2, _FBLK), lambda i: (0, 0)),
                  pl.BlockSpec((2, _FBLK), lambda i: (0, 0)),
                  pl.BlockSpec((_FBLK, _FBLK), lambda i: (0, 0))],
        out_specs=pl.BlockSpec((_FBLK, _FBLK), lambda i: (i, 0)),
        out_shape=jax.ShapeDtypeStruct((_FROWS, _FBLK), f32),
    )(agg0f, deg0f, c0t, s0t, W1big)

    (agg1,) = _sc_agg_nodeg(y1f.reshape(_NROWS, _NQ), edge_index,
                            z2, ones16)
    agg1f = agg1.reshape(2, _FROWS, _FBLK)

    out = pl.pallas_call(
        _fin_body,
        grid=(_NBLK,),
        in_specs=[pl.BlockSpec((2, _FBLK, _FBLK), lambda i: (0, i, 0)),
                  pl.BlockSpec((2, _FBLK, _FBLK), lambda i: (0, i, 0)),
                  pl.BlockSpec((2, _FBLK), lambda i: (0, 0)),
                  pl.BlockSpec((2, _FBLK), lambda i: (0, 0)),
                  pl.BlockSpec((_PACK, _FBLK), lambda i: (0, i)),
                  pl.BlockSpec((_NQ, _OUT), lambda i: (0, 0)),
                  pl.BlockSpec((1, _OUT), lambda i: (0, 0))],
        out_specs=pl.BlockSpec((_N_GRAPHS, _OUT), lambda i: (0, 0)),
        out_shape=jax.ShapeDtypeStruct((_N_GRAPHS, _OUT), f32),
        scratch_shapes=[pltpu.VMEM((_N_GRAPHS, _NQ), f32),
                        pltpu.VMEM((_N_GRAPHS, _NQ), f32)],
    )(agg1f, deg0f, c1t, s1t, batchT, Wc, bc2)

    return out


# speedup vs baseline: 39.7853x; 1.1725x over previous
import jax
import jax.numpy as jnp
from jax import lax
from jax.experimental import pallas as pl
from jax.experimental.pallas import tpu as pltpu
from jax.experimental.pallas import tpu_sc as plsc

_N_NODES = 10000
_N_EDGES = 320000
_D_FEAT = 128
_NQ = 16
_N_GRAPHS = 64
_OUT = 10

_NW = 32
_EPW = _N_EDGES // _NW
_GSZ = 1024
_NFULL = _EPW // _GSZ
_EPI = _EPW - _NFULL * _GSZ
_CH = 128
_CH_PER_G = _GSZ // _CH
_EPI_CH = _EPI // _CH
_TAIL = _EPI - _EPI_CH * _CH
_NROWS = 10240
_RPT = _NROWS // 16

_PACK = 8
_FROWS = _NROWS // _PACK
_FBLK = 128
_TBLK = 640
_NBLK = _FROWS // _TBLK


def _make_sc_agg(with_deg, table_rows):
    def body(y_hbm, ei_hbm, z2_hbm, z1_hbm, ones_hbm, *out_and_scratch):
        if with_deg:
            (agg_out, deg_out, src_v, dst_v, rows_v, ones_v, degc_v,
             deg16_v, acc_s, deg_s, y_s, gsem, ssem) = out_and_scratch
        else:
            (agg_out, src_v, dst_v, rows_v, acc_s, y_s, gsem, ssem) = \
                out_and_scratch
        cid = lax.axis_index("c")
        sid = lax.axis_index("s")
        wid = sid * 2 + cid
        base = wid * _EPW

        ypt = table_rows // 16
        pltpu.sync_copy(z2_hbm, acc_s.at[pl.ds(sid * _RPT, _RPT)])
        pltpu.sync_copy(y_hbm.at[pl.ds(sid * ypt, ypt)],
                        y_s.at[pl.ds(sid * ypt, ypt)])
        if with_deg:
            pltpu.sync_copy(z1_hbm, deg_s.at[pl.ds(sid * _RPT, _RPT)])
            pltpu.sync_copy(ones_hbm, ones_v)
        pltpu.sync_copy(ei_hbm.at[0, pl.ds(base, _EPW)], src_v)
        pltpu.sync_copy(ei_hbm.at[1, pl.ds(base, _EPW)], dst_v)
        plsc.subcore_barrier()

        def scat(eoff, rows, roff, n):
            idx = dst_v.at[pl.ds(eoff, n)]
            pltpu.async_copy(rows.at[pl.ds(roff, n)], acc_s.at[idx],
                             ssem, add=True)
            if with_deg:
                pltpu.async_copy(ones_v.at[pl.ds(0, n)], deg_s.at[idx],
                                 ssem, add=True)

        def scat_wait(eoff, rows, roff, n):
            idx = dst_v.at[pl.ds(eoff, n)]
            pltpu.make_async_copy(rows.at[pl.ds(roff, n)], acc_s.at[idx],
                                  ssem).wait()
            if with_deg:
                pltpu.make_async_copy(ones_v.at[pl.ds(0, n)], deg_s.at[idx],
                                      ssem).wait()

        pltpu.async_copy(y_s.at[src_v.at[pl.ds(0, _GSZ)]], rows_v.at[0],
                         gsem)

        def group(g, carry):
            buf = lax.rem(g, 2)
            rows = rows_v.at[buf]
            pltpu.make_async_copy(y_s.at[src_v.at[pl.ds(0, _GSZ)]], rows,
                                  gsem).wait()

            @pl.when(g + 1 < _NFULL)
            def _():
                pltpu.async_copy(
                    y_s.at[src_v.at[pl.ds((g + 1) * _GSZ, _GSZ)]],
                    rows_v.at[1 - buf], gsem)

            @pl.when(g + 1 == _NFULL)
            def _():
                pltpu.async_copy(
                    y_s.at[src_v.at[pl.ds(_NFULL * _GSZ, _EPI)]],
                    rows_v.at[1 - buf].at[pl.ds(0, _EPI)], gsem)

            for j in range(_CH_PER_G):
                scat(g * _GSZ + j * _CH, rows, j * _CH, _CH)
            for j in range(_CH_PER_G):
                scat_wait(g * _GSZ + j * _CH, rows, j * _CH, _CH)
            return carry

        lax.fori_loop(0, _NFULL, group, 0)
        erows = rows_v.at[_NFULL % 2]
        pltpu.make_async_copy(y_s.at[src_v.at[pl.ds(0, _EPI)]],
                              erows.at[pl.ds(0, _EPI)], gsem).wait()
        ebase = _NFULL * _GSZ
        for j in range(_EPI_CH):
            scat(ebase + j * _CH, erows, j * _CH, _CH)
        scat(ebase + _EPI_CH * _CH, erows, _EPI_CH * _CH, _TAIL)
        for j in range(_EPI_CH):
            scat_wait(ebase + j * _CH, erows, j * _CH, _CH)
        scat_wait(ebase + _EPI_CH * _CH, erows, _EPI_CH * _CH, _TAIL)
        plsc.subcore_barrier()

        pltpu.sync_copy(acc_s.at[pl.ds(sid * _RPT, _RPT)],
                        agg_out.at[cid, pl.ds(sid * _RPT, _RPT)])
        if with_deg:
            pltpu.sync_copy(deg_s.at[pl.ds(sid * _RPT, _RPT)], degc_v)

            def expand(i, carry):
                for k in range(16):
                    splat = plsc.load_gather(
                        degc_v, [jnp.full((16,), i * 16 + k, jnp.int32)])
                    deg16_v[i * 16 + k] = splat
                return carry

            lax.fori_loop(0, _RPT // 16, expand, 0)
            pltpu.sync_copy(deg16_v,
                            deg_out.at[cid, pl.ds(sid * _RPT, _RPT)])

    out_type = [jax.ShapeDtypeStruct((2, _NROWS, _NQ), jnp.float32)]
    scratch = [
        pltpu.VMEM((_EPW,), jnp.int32),
        pltpu.VMEM((_EPW,), jnp.int32),
        pltpu.VMEM((2, _GSZ, _NQ), jnp.float32),
        pltpu.VMEM_SHARED((_NROWS, _NQ), jnp.float32),
        pltpu.VMEM_SHARED((table_rows, _NQ), jnp.float32),
        pltpu.SemaphoreType.DMA,
        pltpu.SemaphoreType.DMA,
    ]
    if with_deg:
        out_type.append(jax.ShapeDtypeStruct((2, _NROWS, _NQ), jnp.float32))
        scratch.insert(3, pltpu.VMEM((_CH,), jnp.float32))
        scratch.insert(4, pltpu.VMEM((_RPT,), jnp.float32))
        scratch.insert(5, pltpu.VMEM((_RPT, _NQ), jnp.float32))
        scratch.insert(7, pltpu.VMEM_SHARED((_NROWS,), jnp.float32))
    return pl.kernel(
        body,
        out_type=tuple(out_type),
        mesh=plsc.VectorSubcoreMesh(core_axis_name="c", subcore_axis_name="s"),
        scratch_types=scratch,
        compiler_params=pltpu.CompilerParams(use_tc_tiling_on_sc=False,
                                             needs_layout_passes=False),
    )


_sc_agg_deg = _make_sc_agg(True, _N_NODES)
_sc_agg_nodeg = _make_sc_agg(False, _NROWS)


def _mm0_body(x2_ref, w_ref, o_ref):
    o_ref[...] = jnp.dot(x2_ref[...], w_ref[...],
                         preferred_element_type=jnp.float32)


def _shift128():
    row = lax.broadcasted_iota(jnp.int32, (_FBLK, _FBLK), 0)
    col = lax.broadcasted_iota(jnp.int32, (_FBLK, _FBLK), 1)
    same_blk = (row // _NQ) == (col // _NQ)
    rolled = ((row % _NQ) + 1) % _NQ == (col % _NQ)
    return (same_blk & rolled).astype(jnp.float32)


def _post(aggp, degp, c, s):
    a = aggp[0] + aggp[1]
    d = degp[0] + degp[1]
    h = jnp.tanh(a / jnp.maximum(d, 1.0))
    S = _shift128()
    for dd in range(2):
        h = (c[dd][None, :] * h
             + s[dd][None, :] * jnp.dot(h, S,
                                        preferred_element_type=jnp.float32))
    return jnp.where(h >= 0, h, 0.2 * h)


def _mid_body(aggp_ref, degp_ref, c_ref, s_ref, w_ref, o_ref):
    h = _post(aggp_ref[...], degp_ref[...], c_ref[...], s_ref[...])
    o_ref[...] = jnp.dot(h, w_ref[...], preferred_element_type=jnp.float32)


def _fin_body(aggp_ref, degp_ref, c_ref, s_ref, bt_ref, wc_ref, bc_ref,
              o_ref, sums, cnts):
    i = pl.program_id(0)
    h = _post(aggp_ref[...], degp_ref[...], c_ref[...], s_ref[...])
    bt = bt_ref[...]
    giota = lax.broadcasted_iota(jnp.int32, (_N_GRAPHS, _TBLK), 0)
    psum = jnp.zeros((_N_GRAPHS, _NQ), jnp.float32)
    osum = jnp.zeros((_N_GRAPHS, _TBLK), jnp.float32)
    for k in range(_PACK):
        onehot = (giota == bt[k][None, :]).astype(jnp.float32)
        hk = h[:, k * _NQ:(k + 1) * _NQ]
        psum += jnp.dot(onehot, hk, preferred_element_type=jnp.float32)
        osum += onehot
    pcnt = jnp.dot(osum, jnp.ones((_TBLK, _NQ), jnp.float32),
                   preferred_element_type=jnp.float32)

    @pl.when(i == 0)
    def _():
        sums[...] = jnp.zeros_like(sums)
        cnts[...] = jnp.zeros_like(cnts)

    sums[...] += psum
    cnts[...] += pcnt

    @pl.when(i == _NBLK - 1)
    def _():
        pooled = sums[...] / jnp.maximum(cnts[...], 1.0)
        o_ref[...] = (jnp.dot(pooled, wc_ref[...],
                              preferred_element_type=jnp.float32)
                      + bc_ref[...])


def _blockdiag(w):
    return jnp.kron(jnp.eye(_PACK, dtype=jnp.float32), w)


def kernel(x, edge_index, batch, W0, theta0, W1, theta1, Wc, bc):
    f32 = jnp.float32

    z2 = jnp.zeros((_RPT, _NQ), f32)
    z1 = jnp.zeros((_RPT,), f32)
    ones1 = jnp.ones((_CH,), f32)
    c0t = jnp.tile(jnp.cos(theta0), (1, _PACK))
    s0t = jnp.tile(jnp.sin(theta0), (1, _PACK))
    c1t = jnp.tile(jnp.cos(theta1), (1, _PACK))
    s1t = jnp.tile(jnp.sin(theta1), (1, _PACK))
    W0big = jnp.kron(jnp.eye(_PACK, dtype=f32), W0)
    W1big = _blockdiag(W1)
    batch_pad = jnp.concatenate(
        [batch, jnp.full((_NROWS - _N_NODES,), _N_GRAPHS, jnp.int32)])
    batchT = batch_pad.reshape(_FROWS, _PACK).T
    bc2 = bc.reshape(1, _OUT)
    x2 = x.reshape(_N_NODES // _PACK, _PACK * _D_FEAT)

    y0f = pl.pallas_call(
        _mm0_body,
        in_specs=[pl.BlockSpec(x2.shape, lambda: (0, 0)),
                  pl.BlockSpec(W0big.shape, lambda: (0, 0))],
        out_specs=pl.BlockSpec((_N_NODES // _PACK, _FBLK), lambda: (0, 0)),
        out_shape=jax.ShapeDtypeStruct((_N_NODES // _PACK, _FBLK), f32),
    )(x2, W0big)

    agg0, deg0 = _sc_agg_deg(y0f.reshape(_N_NODES, _NQ), edge_index,
                             z2, z1, ones1)
    agg0f = agg0.reshape(2, _FROWS, _FBLK)
    deg0f = deg0.reshape(2, _FROWS, _FBLK)

    y1f = pl.pallas_call(
        _mid_body,
        grid=(_NBLK,),
        in_specs=[pl.BlockSpec((2, _TBLK, _FBLK), lambda i: (0, i, 0)),
                  pl.BlockSpec((2, _TBLK, _FBLK), lambda i: (0, i, 0)),
                  pl.BlockSpec((2, _FBLK), lambda i: (0, 0)),
                  pl.BlockSpec((2, _FBLK), lambda i: (0, 0)),
                  pl.BlockSpec((_FBLK, _FBLK), lambda i: (0, 0))],
        out_specs=pl.BlockSpec((_TBLK, _FBLK), lambda i: (i, 0)),
        out_shape=jax.ShapeDtypeStruct((_FROWS, _FBLK), f32),
    )(agg0f, deg0f, c0t, s0t, W1big)

    (agg1,) = _sc_agg_nodeg(y1f.reshape(_NROWS, _NQ), edge_index,
                            z2, z1, ones1)
    agg1f = agg1.reshape(2, _FROWS, _FBLK)

    out = pl.pallas_call(
        _fin_body,
        grid=(_NBLK,),
        in_specs=[pl.BlockSpec((2, _TBLK, _FBLK), lambda i: (0, i, 0)),
                  pl.BlockSpec((2, _TBLK, _FBLK), lambda i: (0, i, 0)),
                  pl.BlockSpec((2, _FBLK), lambda i: (0, 0)),
                  pl.BlockSpec((2, _FBLK), lambda i: (0, 0)),
                  pl.BlockSpec((_PACK, _TBLK), lambda i: (0, i)),
                  pl.BlockSpec((_NQ, _OUT), lambda i: (0, 0)),
                  pl.BlockSpec((1, _OUT), lambda i: (0, 0))],
        out_specs=pl.BlockSpec((_N_GRAPHS, _OUT), lambda i: (0, 0)),
        out_shape=jax.ShapeDtypeStruct((_N_GRAPHS, _OUT), f32),
        scratch_shapes=[pltpu.VMEM((_N_GRAPHS, _NQ), f32),
                        pltpu.VMEM((_N_GRAPHS, _NQ), f32)],
    )(agg1f, deg0f, c1t, s1t, batchT, Wc, bc2)

    return out
